# baseline (speedup 1.0000x reference)
import functools

import jax
import jax.numpy as jnp
from jax import lax
from jax.experimental import pallas as pl
from jax.experimental.pallas import tpu as pltpu
from jax.experimental.pallas import tpu_sc as plsc

N = 10000
E = 160000
D = 128
NSTEPS = 16
B = 128

NC, NS = 2, 16
NW = NC * NS
EPW = E // NW
KE = 40
NCH = EPW // KE
SK = 5
KZ = 200
NZCH = N // KZ
KG = 40
NGCH = N // KG
KH = 16

_f32 = jnp.float32


@functools.cache
def _mesh():
    return plsc.VectorSubcoreMesh(core_axis_name="c", subcore_axis_name="s")


def _zero_acc_start(zeros_hbm, s_sh, s, zsem):
    def cp(j, _):
        k = s + NS * j

        @pl.when(k < NZCH)
        def _():
            off = k * KZ
            pltpu.async_copy(zeros_hbm.at[pl.ds(off, KZ)],
                             s_sh.at[pl.ds(off, KZ)], zsem)

        return 0

    lax.fori_loop(0, (NZCH + NS - 1) // NS, cp, 0)


def _zero_acc_wait(zeros_hbm, s_sh, s, zsem):
    def cp(j, _):
        k = s + NS * j

        @pl.when(k < NZCH)
        def _():
            off = k * KZ
            pltpu.make_async_copy(zeros_hbm.at[pl.ds(off, KZ)],
                                  s_sh.at[pl.ds(off, KZ)], zsem).wait()

        return 0

    lax.fori_loop(0, (NZCH + NS - 1) // NS, cp, 0)


def _flush_acc(s_sh, out_hbm, c, s, zsem):
    def cp(issue, j, _):
        k = s + NS * j

        @pl.when(k < NZCH)
        def _():
            off = k * KZ
            d = pltpu.make_async_copy(s_sh.at[pl.ds(off, KZ)],
                                      out_hbm.at[c, pl.ds(off, KZ)], zsem)
            if issue:
                d.start()
            else:
                d.wait()

        return 0

    lax.fori_loop(0, (NZCH + NS - 1) // NS, functools.partial(cp, True), 0)
    lax.fori_loop(0, (NZCH + NS - 1) // NS, functools.partial(cp, False), 0)


@functools.cache
def _edge_pass_kernel():
    return functools.partial(
        pl.kernel,
        out_type=jax.ShapeDtypeStruct((NC, N, D), _f32),
        mesh=_mesh(),
        scratch_types=(
            [pltpu.VMEM((SK * KE,), jnp.int32)] * 4
            + [pltpu.VMEM((KE, D), _f32)] * 9
            + [pltpu.VMEM_SHARED((N, D), _f32)]
            + [pltpu.SemaphoreType.DMA] * 10
        ),
    )(_edge_pass_body)


def _edge_pass_body(p_hbm, q_hbm, src_hbm, dst_hbm, zeros_hbm, out_hbm,
                    is0, is1, id0, id1,
                    pr0, pr1, qr0, qr1, sc0, sc1, sc2, sc3, sc4,
                    s_sh, si0, si1, gs0, gs1,
                    ss0, ss1, ss2, ss3, ss4, zsem):
    c = lax.axis_index("c")
    s = lax.axis_index("s")
    w = c * NS + s

    _zero_acc_start(zeros_hbm, s_sh, s, zsem)

    base_w = w * EPW
    idx_s, idx_d = (is0, is1), (id0, id1)
    pr, qr = (pr0, pr1), (qr0, qr1)
    scb = (sc0, sc1, sc2, sc3, sc4)
    isem = (si0, si1)
    gsem = (gs0, gs1)
    ssem = (ss0, ss1, ss2, ss3, ss4)

    def load_idx(ci0, m):
        base = base_w + ci0 * KE
        pltpu.async_copy(src_hbm.at[pl.ds(base, SK * KE)], idx_s[m], isem[m])
        pltpu.async_copy(dst_hbm.at[pl.ds(base, SK * KE)], idx_d[m], isem[m])

    def wait_idx(m):
        pltpu.make_async_copy(src_hbm.at[pl.ds(0, SK * KE)], idx_s[m],
                              isem[m]).wait()
        pltpu.make_async_copy(src_hbm.at[pl.ds(0, SK * KE)], idx_d[m],
                              isem[m]).wait()

    def gather_start(m, row, k2):
        sl = pl.ds(row * KE, KE)
        pltpu.async_copy(p_hbm.at[idx_s[m].at[sl]], pr[k2], gsem[k2])
        pltpu.async_copy(q_hbm.at[idx_d[m].at[sl]], qr[k2], gsem[k2])

    def wait_gathers(k2):
        pltpu.make_async_copy(p_hbm.at[pl.ds(0, KE)], pr[k2], gsem[k2]).wait()
        pltpu.make_async_copy(q_hbm.at[pl.ds(0, KE)], qr[k2], gsem[k2]).wait()

    def drain_scatter(k5):
        pltpu.make_async_copy(p_hbm.at[pl.ds(0, KE)], scb[k5], ssem[k5]).wait()

    def compute(k5, k2):
        def row(r, _):
            r0 = 2 * r
            for rr in (r0, r0 + 1):
                for cc in range(D // 16):
                    sl = pl.ds(cc * 16, 16)
                    scb[k5][rr, sl] = jnp.maximum(
                        pr[k2][rr, sl] + qr[k2][rr, sl], 0.0)
            return 0

        lax.fori_loop(0, KE // 2, row, 0)

    def scatter_start(k5, m, row):
        pltpu.async_copy(scb[k5], s_sh.at[idx_s[m].at[pl.ds(row * KE, KE)]],
                         ssem[k5], add=True)

    load_idx(0, 0)
    wait_idx(0)
    gather_start(0, 0, 0)
    _zero_acc_wait(zeros_hbm, s_sh, s, zsem)
    plsc.subcore_barrier()

    def piped(g, _):
        for jj in range(2 * SK):
            ci = 10 * g + jj
            m, row = divmod(jj, SK)
            k2, k5 = jj % 2, jj % 5

            @pl.when(ci >= 2)
            def _():
                drain_scatter((k5 - 2) % 5)

            if jj == 2:
                load_idx(10 * g + 5, 1)
            if jj == 7:
                load_idx(10 * g + 10, 0)
            if jj == 4:
                wait_idx(1)
            if jj == 9:
                wait_idx(0)
            m1, row1 = divmod((jj + 1) % (2 * SK), SK)
            gather_start(m1, row1, (k2 + 1) % 2)
            wait_gathers(k2)
            compute(k5, k2)
            scatter_start(k5, m, row)
        return 0

    nmain = NCH // (2 * SK)
    lax.fori_loop(0, nmain, piped, 0)

    for jj in range(SK):
        ci = 10 * nmain + jj
        k2, k5 = jj % 2, jj % 5
        drain_scatter((k5 - 2) % 5)
        if jj + 1 < SK:
            gather_start(0, jj + 1, (k2 + 1) % 2)
        wait_gathers(k2)
        compute(k5, k2)
        scatter_start(k5, 0, jj)
    drain_scatter((NCH - 2) % 5)
    drain_scatter((NCH - 1) % 5)

    plsc.subcore_barrier()
    _flush_acc(s_sh, out_hbm, c, s, zsem)


@functools.cache
def _deg_pass_kernel():
    return functools.partial(
        pl.kernel,
        out_type=jax.ShapeDtypeStruct((NC, N, D), _f32),
        mesh=_mesh(),
        scratch_types=[
            pltpu.VMEM((KE,), jnp.int32),
            pltpu.VMEM((KE, D), _f32),
            pltpu.VMEM_SHARED((N, D), _f32),
            pltpu.SemaphoreType.DMA,
        ],
    )(_deg_pass_body)


def _deg_pass_body(src_hbm, zeros_hbm, out_hbm, idx_s, ones_b, s_sh, zsem):
    c = lax.axis_index("c")
    s = lax.axis_index("s")
    w = c * NS + s

    _zero_acc_start(zeros_hbm, s_sh, s, zsem)
    one = jnp.full((16,), 1.0, _f32)

    def row(r, _):
        for cc in range(D // 16):
            ones_b[r, pl.ds(cc * 16, 16)] = one
        return 0

    lax.fori_loop(0, KE, row, 0)
    _zero_acc_wait(zeros_hbm, s_sh, s, zsem)
    plsc.subcore_barrier()

    base_w = w * EPW

    def chunk(i, _):
        base = base_w + i * KE
        pltpu.sync_copy(src_hbm.at[pl.ds(base, KE)], idx_s)
        pltpu.sync_copy(ones_b, s_sh.at[idx_s], add=True)
        return 0

    lax.fori_loop(0, NCH, chunk, 0)
    plsc.subcore_barrier()
    _flush_acc(s_sh, out_hbm, c, s, zsem)


@functools.cache
def _gather3_kernel():
    return functools.partial(
        pl.kernel,
        out_type=[jax.ShapeDtypeStruct((N, D), _f32)] * 3,
        mesh=_mesh(),
        scratch_types=[
            pltpu.VMEM((KG,), jnp.int32),
            pltpu.VMEM((KG, D), _f32),
            pltpu.SemaphoreType.DMA,
        ],
    )(_gather3_body)


def _gather3_body(t1, t2, t3, idx_hbm, o1, o2, o3, idxv, buf, sem):
    c = lax.axis_index("c")
    s = lax.axis_index("s")
    w = c * NS + s

    def rnd(i, _):
        cid = w + i * NW

        @pl.when(cid < NGCH)
        def _():
            base = cid * KG
            pltpu.sync_copy(idx_hbm.at[pl.ds(base, KG)], idxv)
            for tbl, out in ((t1, o1), (t2, o2), (t3, o3)):
                pltpu.async_copy(tbl.at[idxv], buf, sem).wait()
                pltpu.sync_copy(buf, out.at[pl.ds(base, KG)])

        return 0

    lax.fori_loop(0, (NGCH + NW - 1) // NW, rnd, 0)


@functools.cache
def _heads_gather_kernel():
    return functools.partial(
        pl.kernel,
        out_type=[jax.ShapeDtypeStruct((B, D), _f32)] * 2,
        mesh=_mesh(),
        scratch_types=[
            pltpu.VMEM((KH,), jnp.int32),
            pltpu.VMEM((KH, D), _f32),
            pltpu.SemaphoreType.DMA,
        ],
    )(_heads_gather_body)


def _heads_gather_body(xa, ha, xb, hb, oa, ob, idxv, buf, sem):
    c = lax.axis_index("c")
    s = lax.axis_index("s")
    base = s * KH
    active = s < (B // KH)

    @pl.when(jnp.logical_and(c == 0, active))
    def _():
        pltpu.sync_copy(ha.at[pl.ds(base, KH)], idxv)
        pltpu.async_copy(xa.at[idxv], buf, sem).wait()
        pltpu.sync_copy(buf, oa.at[pl.ds(base, KH)])

    @pl.when(jnp.logical_and(c == 1, active))
    def _():
        pltpu.sync_copy(hb.at[pl.ds(base, KH)], idxv)
        pltpu.async_copy(xb.at[idxv], buf, sem).wait()
        pltpu.sync_copy(buf, ob.at[pl.ds(base, KH)])


def _prep_body(emb_r, w1a_r, w1b_r, b1_r, pw1b_r, pb1_r, e1_r, e2_r, e3_r):
    x = emb_r[...]
    e1_r[...] = jnp.dot(x, w1a_r[...], preferred_element_type=_f32)
    e2_r[...] = jnp.dot(x, w1b_r[...], preferred_element_type=_f32) + b1_r[...]
    e3_r[...] = jnp.dot(x, pw1b_r[...], preferred_element_type=_f32) + pb1_r[...]


def _step_body(s_r, sd_r, c0_r, w2_r, b2_r, pw1a_r, pw2_r, pb2_r,
               w1a_r, w1b_r, b1_r, p_r, q_r, x_r):
    S = s_r[0] + s_r[1]
    deg = sd_r[0] + sd_r[1]
    agg = jnp.dot(S, w2_r[...], preferred_element_type=_f32) + deg * b2_r[...]
    t = jnp.maximum(jnp.dot(agg, pw1a_r[...], preferred_element_type=_f32)
                    + c0_r[...], 0.0)
    x = jnp.dot(t, pw2_r[...], preferred_element_type=_f32) + pb2_r[...]
    x_r[...] = x
    p_r[...] = jnp.dot(x, w1a_r[...], preferred_element_type=_f32)
    q_r[...] = jnp.dot(x, w1b_r[...], preferred_element_type=_f32) + b1_r[...]


def _logits_body(ea_r, eb_r, l1a_r, l1b_r, lb1_r, l2_r, lb2_r, o_r):
    h = jnp.maximum(jnp.dot(ea_r[...], l1a_r[...], preferred_element_type=_f32)
                    + jnp.dot(eb_r[...], l1b_r[...], preferred_element_type=_f32)
                    + lb1_r[...], 0.0)
    o_r[...] = jnp.dot(h, l2_r[...], preferred_element_type=_f32) + lb2_r[...]


_BM = 1000


def _tc_step(S, Sd, C0, w2, b2, pw1a, pw2, pb2, w1a, w1b, b1):
    full2 = pl.BlockSpec((D, D), lambda i: (0, 0))
    bias = pl.BlockSpec((1, D), lambda i: (0, 0))
    blk = pl.BlockSpec((_BM, D), lambda i: (i, 0))
    acc = pl.BlockSpec((NC, _BM, D), lambda i: (0, i, 0))
    return pl.pallas_call(
        _step_body,
        grid=(N // _BM,),
        in_specs=[acc, acc, blk, full2, bias, full2, full2, bias,
                  full2, full2, bias],
        out_specs=[blk, blk, blk],
        out_shape=[jax.ShapeDtypeStruct((N, D), _f32)] * 3,
    )(S, Sd, C0, w2, b2, pw1a, pw2, pb2, w1a, w1b, b1)


def kernel(nodes_a, edges_a, heads_a, nodes_b, edges_b, heads_b,
           emb, msg_W1, msg_b1, msg_W2, msg_b2,
           post_W1, post_b1, post_W2, post_b2,
           logit_W1, logit_b1, logit_W2, logit_b2):
    i32 = jnp.int32
    w1a, w1b = msg_W1[:D], msg_W1[D:2 * D]
    pw1a, pw1b = post_W1[:D], post_W1[D:]
    l1a, l1b = logit_W1[:D], logit_W1[D:]
    b1 = msg_b1.reshape(1, D)
    b2 = msg_b2.reshape(1, D)
    pb1 = post_b1.reshape(1, D)
    pb2 = post_b2.reshape(1, D)
    lb1 = logit_b1.reshape(1, D)
    lb2 = jnp.broadcast_to(logit_b2.reshape(1, 1), (1, D))
    l2p = jnp.pad(logit_W2, ((0, 0), (0, D - 1)))

    e1, e2, e3 = pl.pallas_call(
        _prep_body,
        out_shape=[jax.ShapeDtypeStruct((emb.shape[0], D), _f32)] * 3,
    )(emb, w1a, w1b, b1, pw1b, pb1)

    def encode(nodes, edges, heads_unused):
        nodes = nodes.astype(i32)
        src = edges[:, 0].astype(i32)
        dst = edges[:, 1].astype(i32)
        zeros = jnp.zeros((N, D), _f32)
        P, Q, C0 = _gather3_kernel()(e1, e2, e3, nodes)
        Sd = _deg_pass_kernel()(src, zeros)
        x = None
        for _ in range(NSTEPS):
            S = _edge_pass_kernel()(P, Q, src, dst, zeros)
            P, Q, x = _tc_step(S, Sd, C0, msg_W2, b2, pw1a, post_W2, pb2,
                               w1a, w1b, b1)
        return x

    xa = encode(nodes_a, edges_a, heads_a)
    xb = encode(nodes_b, edges_b, heads_b)
    ea, eb = _heads_gather_kernel()(xa, heads_a.astype(i32),
                                    xb, heads_b.astype(i32))

    out = pl.pallas_call(
        _logits_body,
        out_shape=jax.ShapeDtypeStruct((B, D), _f32),
    )(ea, eb, l1a, l1b, lb1, l2p, lb2)
    return out[:, :1]


# speedup vs baseline: 1.0140x; 1.0140x over previous
import functools

import jax
import jax.numpy as jnp
from jax import lax
from jax.experimental import pallas as pl
from jax.experimental.pallas import tpu as pltpu
from jax.experimental.pallas import tpu_sc as plsc

N = 10000
E = 160000
D = 128
NSTEPS = 16
B = 128

NC, NS = 2, 16
NW = NC * NS
EPW = E // NW
KE = 40
NCH = EPW // KE
KZ = 200
NZCH = N // KZ
KG = 40
NGCH = N // KG
KH = 16

_f32 = jnp.float32


@functools.cache
def _mesh():
    return plsc.VectorSubcoreMesh(core_axis_name="c", subcore_axis_name="s")


def _zero_acc_start(zeros_hbm, s_sh, s, zsem):
    def cp(j, _):
        k = s + NS * j

        @pl.when(k < NZCH)
        def _():
            off = k * KZ
            pltpu.async_copy(zeros_hbm.at[pl.ds(off, KZ)],
                             s_sh.at[pl.ds(off, KZ)], zsem)

        return 0

    lax.fori_loop(0, (NZCH + NS - 1) // NS, cp, 0)


def _zero_acc_wait(zeros_hbm, s_sh, s, zsem):
    def cp(j, _):
        k = s + NS * j

        @pl.when(k < NZCH)
        def _():
            off = k * KZ
            pltpu.make_async_copy(zeros_hbm.at[pl.ds(off, KZ)],
                                  s_sh.at[pl.ds(off, KZ)], zsem).wait()

        return 0

    lax.fori_loop(0, (NZCH + NS - 1) // NS, cp, 0)


def _flush_acc(s_sh, out_hbm, c, s, zsem):
    def cp(issue, j, _):
        k = s + NS * j

        @pl.when(k < NZCH)
        def _():
            off = k * KZ
            d = pltpu.make_async_copy(s_sh.at[pl.ds(off, KZ)],
                                      out_hbm.at[c, pl.ds(off, KZ)], zsem)
            if issue:
                d.start()
            else:
                d.wait()

        return 0

    lax.fori_loop(0, (NZCH + NS - 1) // NS, functools.partial(cp, True), 0)
    lax.fori_loop(0, (NZCH + NS - 1) // NS, functools.partial(cp, False), 0)


@functools.cache
def _edge_pass_kernel():
    return functools.partial(
        pl.kernel,
        out_type=jax.ShapeDtypeStruct((NC, N, D), _f32),
        mesh=_mesh(),
        scratch_types=(
            [pltpu.VMEM((KE,), jnp.int32)] * 8
            + [pltpu.VMEM((KE, D), _f32)] * 8
            + [pltpu.VMEM_SHARED((N, D), _f32)]
            + [pltpu.SemaphoreType.DMA] * 11
        ),
    )(_edge_pass_body)


def _edge_pass_body(p_hbm, q_hbm, src_hbm, dst_hbm, zeros_hbm, out_hbm,
                    is0, is1, is2, is3, id0, id1, id2, id3,
                    pr0, pr1, qr0, qr1, sc0, sc1, sc2, sc3,
                    s_sh, si0, si1, si2, si3, gs0, gs1,
                    ss0, ss1, ss2, ss3, zsem):
    c = lax.axis_index("c")
    s = lax.axis_index("s")
    w = c * NS + s

    _zero_acc_start(zeros_hbm, s_sh, s, zsem)

    base_w = w * EPW
    idx_s, idx_d = (is0, is1, is2, is3), (id0, id1, id2, id3)
    pr, qr, scb = (pr0, pr1), (qr0, qr1), (sc0, sc1, sc2, sc3)
    isem = (si0, si1, si2, si3)
    gsem, ssem = (gs0, gs1), (ss0, ss1, ss2, ss3)

    def load_idx(ci, k4):
        base = base_w + ci * KE
        pltpu.async_copy(src_hbm.at[pl.ds(base, KE)], idx_s[k4], isem[k4])
        pltpu.async_copy(dst_hbm.at[pl.ds(base, KE)], idx_d[k4], isem[k4])

    def wait_idx(k4):
        pltpu.make_async_copy(src_hbm.at[pl.ds(0, KE)], idx_s[k4], isem[k4]).wait()
        pltpu.make_async_copy(src_hbm.at[pl.ds(0, KE)], idx_d[k4], isem[k4]).wait()

    def gather_start(k4, k2):
        pltpu.async_copy(p_hbm.at[idx_s[k4]], pr[k2], gsem[k2])
        pltpu.async_copy(q_hbm.at[idx_d[k4]], qr[k2], gsem[k2])

    def wait_gathers(k2):
        pltpu.make_async_copy(p_hbm.at[pl.ds(0, KE)], pr[k2], gsem[k2]).wait()
        pltpu.make_async_copy(q_hbm.at[pl.ds(0, KE)], qr[k2], gsem[k2]).wait()

    def drain_scatter(k4):
        pltpu.make_async_copy(p_hbm.at[pl.ds(0, KE)], scb[k4], ssem[k4]).wait()

    def compute(k4, k2):
        def row(r, _):
            r0 = 2 * r
            for rr in (r0, r0 + 1):
                for cc in range(D // 16):
                    sl = pl.ds(cc * 16, 16)
                    scb[k4][rr, sl] = jnp.maximum(
                        pr[k2][rr, sl] + qr[k2][rr, sl], 0.0)
            return 0

        lax.fori_loop(0, KE // 2, row, 0)

    def scatter_start(k4):
        pltpu.async_copy(scb[k4], s_sh.at[idx_s[k4]], ssem[k4], add=True)

    load_idx(0, 0)
    load_idx(1, 1)
    wait_idx(0)
    gather_start(0, 0)
    _zero_acc_wait(zeros_hbm, s_sh, s, zsem)
    plsc.subcore_barrier()

    def piped(g, _):
        for b in range(4):
            ci = 4 * g + b
            k4, k2 = b, b % 2

            @pl.when(ci >= 2)
            def _():
                drain_scatter((k4 + 2) % 4)

            @pl.when(ci < NCH - 2)
            def _():
                load_idx(ci + 2, (k4 + 2) % 4)

            wait_idx((k4 + 1) % 4)
            gather_start((k4 + 1) % 4, (k2 + 1) % 2)
            wait_gathers(k2)
            compute(k4, k2)
            scatter_start(k4)
        return 0

    lax.fori_loop(0, NCH // 4, piped, 0)

    ci = NCH - 1
    k4, k2 = ci % 4, ci % 2
    drain_scatter((k4 + 2) % 4)
    wait_gathers(k2)
    compute(k4, k2)
    scatter_start(k4)
    drain_scatter((NCH - 2) % 4)
    drain_scatter((NCH - 1) % 4)

    plsc.subcore_barrier()
    _flush_acc(s_sh, out_hbm, c, s, zsem)


@functools.cache
def _deg_pass_kernel():
    return functools.partial(
        pl.kernel,
        out_type=jax.ShapeDtypeStruct((NC, N, D), _f32),
        mesh=_mesh(),
        scratch_types=[
            pltpu.VMEM((KE,), jnp.int32),
            pltpu.VMEM((KE, D), _f32),
            pltpu.VMEM_SHARED((N, D), _f32),
            pltpu.SemaphoreType.DMA,
        ],
    )(_deg_pass_body)


def _deg_pass_body(src_hbm, zeros_hbm, out_hbm, idx_s, ones_b, s_sh, zsem):
    c = lax.axis_index("c")
    s = lax.axis_index("s")
    w = c * NS + s

    _zero_acc_start(zeros_hbm, s_sh, s, zsem)
    one = jnp.full((16,), 1.0, _f32)

    def row(r, _):
        for cc in range(D // 16):
            ones_b[r, pl.ds(cc * 16, 16)] = one
        return 0

    lax.fori_loop(0, KE, row, 0)
    _zero_acc_wait(zeros_hbm, s_sh, s, zsem)
    plsc.subcore_barrier()

    base_w = w * EPW

    def chunk(i, _):
        base = base_w + i * KE
        pltpu.sync_copy(src_hbm.at[pl.ds(base, KE)], idx_s)
        pltpu.sync_copy(ones_b, s_sh.at[idx_s], add=True)
        return 0

    lax.fori_loop(0, NCH, chunk, 0)
    plsc.subcore_barrier()
    _flush_acc(s_sh, out_hbm, c, s, zsem)


@functools.cache
def _gather3_kernel():
    return functools.partial(
        pl.kernel,
        out_type=[jax.ShapeDtypeStruct((N, D), _f32)] * 3,
        mesh=_mesh(),
        scratch_types=[
            pltpu.VMEM((KG,), jnp.int32),
            pltpu.VMEM((KG, D), _f32),
            pltpu.SemaphoreType.DMA,
        ],
    )(_gather3_body)


def _gather3_body(t1, t2, t3, idx_hbm, o1, o2, o3, idxv, buf, sem):
    c = lax.axis_index("c")
    s = lax.axis_index("s")
    w = c * NS + s

    def rnd(i, _):
        cid = w + i * NW

        @pl.when(cid < NGCH)
        def _():
            base = cid * KG
            pltpu.sync_copy(idx_hbm.at[pl.ds(base, KG)], idxv)
            for tbl, out in ((t1, o1), (t2, o2), (t3, o3)):
                pltpu.async_copy(tbl.at[idxv], buf, sem).wait()
                pltpu.sync_copy(buf, out.at[pl.ds(base, KG)])

        return 0

    lax.fori_loop(0, (NGCH + NW - 1) // NW, rnd, 0)


@functools.cache
def _heads_gather_kernel():
    return functools.partial(
        pl.kernel,
        out_type=[jax.ShapeDtypeStruct((B, D), _f32)] * 2,
        mesh=_mesh(),
        scratch_types=[
            pltpu.VMEM((KH,), jnp.int32),
            pltpu.VMEM((KH, D), _f32),
            pltpu.SemaphoreType.DMA,
        ],
    )(_heads_gather_body)


def _heads_gather_body(xa, ha, xb, hb, oa, ob, idxv, buf, sem):
    c = lax.axis_index("c")
    s = lax.axis_index("s")
    base = s * KH
    active = s < (B // KH)

    @pl.when(jnp.logical_and(c == 0, active))
    def _():
        pltpu.sync_copy(ha.at[pl.ds(base, KH)], idxv)
        pltpu.async_copy(xa.at[idxv], buf, sem).wait()
        pltpu.sync_copy(buf, oa.at[pl.ds(base, KH)])

    @pl.when(jnp.logical_and(c == 1, active))
    def _():
        pltpu.sync_copy(hb.at[pl.ds(base, KH)], idxv)
        pltpu.async_copy(xb.at[idxv], buf, sem).wait()
        pltpu.sync_copy(buf, ob.at[pl.ds(base, KH)])


def _prep_body(emb_r, w1a_r, w1b_r, b1_r, pw1b_r, pb1_r, e1_r, e2_r, e3_r):
    x = emb_r[...]
    e1_r[...] = jnp.dot(x, w1a_r[...], preferred_element_type=_f32)
    e2_r[...] = jnp.dot(x, w1b_r[...], preferred_element_type=_f32) + b1_r[...]
    e3_r[...] = jnp.dot(x, pw1b_r[...], preferred_element_type=_f32) + pb1_r[...]


def _step_body(s_r, sd_r, c0_r, w2_r, b2_r, pw1a_r, pw2_r, pb2_r,
               w1a_r, w1b_r, b1_r, p_r, q_r, x_r):
    S = s_r[0] + s_r[1]
    deg = sd_r[0] + sd_r[1]
    agg = jnp.dot(S, w2_r[...], preferred_element_type=_f32) + deg * b2_r[...]
    t = jnp.maximum(jnp.dot(agg, pw1a_r[...], preferred_element_type=_f32)
                    + c0_r[...], 0.0)
    x = jnp.dot(t, pw2_r[...], preferred_element_type=_f32) + pb2_r[...]
    x_r[...] = x
    p_r[...] = jnp.dot(x, w1a_r[...], preferred_element_type=_f32)
    q_r[...] = jnp.dot(x, w1b_r[...], preferred_element_type=_f32) + b1_r[...]


def _logits_body(ea_r, eb_r, l1a_r, l1b_r, lb1_r, l2_r, lb2_r, o_r):
    h = jnp.maximum(jnp.dot(ea_r[...], l1a_r[...], preferred_element_type=_f32)
                    + jnp.dot(eb_r[...], l1b_r[...], preferred_element_type=_f32)
                    + lb1_r[...], 0.0)
    o_r[...] = jnp.dot(h, l2_r[...], preferred_element_type=_f32) + lb2_r[...]


_BM = 1000


def _tc_step(S, Sd, C0, w2, b2, pw1a, pw2, pb2, w1a, w1b, b1):
    full2 = pl.BlockSpec((D, D), lambda i: (0, 0))
    bias = pl.BlockSpec((1, D), lambda i: (0, 0))
    blk = pl.BlockSpec((_BM, D), lambda i: (i, 0))
    acc = pl.BlockSpec((NC, _BM, D), lambda i: (0, i, 0))
    return pl.pallas_call(
        _step_body,
        grid=(N // _BM,),
        in_specs=[acc, acc, blk, full2, bias, full2, full2, bias,
                  full2, full2, bias],
        out_specs=[blk, blk, blk],
        out_shape=[jax.ShapeDtypeStruct((N, D), _f32)] * 3,
    )(S, Sd, C0, w2, b2, pw1a, pw2, pb2, w1a, w1b, b1)


def kernel(nodes_a, edges_a, heads_a, nodes_b, edges_b, heads_b,
           emb, msg_W1, msg_b1, msg_W2, msg_b2,
           post_W1, post_b1, post_W2, post_b2,
           logit_W1, logit_b1, logit_W2, logit_b2):
    i32 = jnp.int32
    w1a, w1b = msg_W1[:D], msg_W1[D:2 * D]
    pw1a, pw1b = post_W1[:D], post_W1[D:]
    l1a, l1b = logit_W1[:D], logit_W1[D:]
    b1 = msg_b1.reshape(1, D)
    b2 = msg_b2.reshape(1, D)
    pb1 = post_b1.reshape(1, D)
    pb2 = post_b2.reshape(1, D)
    lb1 = logit_b1.reshape(1, D)
    lb2 = jnp.broadcast_to(logit_b2.reshape(1, 1), (1, D))
    l2p = jnp.pad(logit_W2, ((0, 0), (0, D - 1)))

    e1, e2, e3 = pl.pallas_call(
        _prep_body,
        out_shape=[jax.ShapeDtypeStruct((emb.shape[0], D), _f32)] * 3,
    )(emb, w1a, w1b, b1, pw1b, pb1)

    def encode(nodes, edges, heads_unused):
        nodes = nodes.astype(i32)
        src = edges[:, 0].astype(i32)
        dst = edges[:, 1].astype(i32)
        zeros = jnp.zeros((N, D), _f32)
        P, Q, C0 = _gather3_kernel()(e1, e2, e3, nodes)
        Sd = _deg_pass_kernel()(src, zeros)
        x = None
        for _ in range(NSTEPS):
            S = _edge_pass_kernel()(P, Q, src, dst, zeros)
            P, Q, x = _tc_step(S, Sd, C0, msg_W2, b2, pw1a, post_W2, pb2,
                               w1a, w1b, b1)
        return x

    xa = encode(nodes_a, edges_a, heads_a)
    xb = encode(nodes_b, edges_b, heads_b)
    ea, eb = _heads_gather_kernel()(xa, heads_a.astype(i32),
                                    xb, heads_b.astype(i32))

    out = pl.pallas_call(
        _logits_body,
        out_shape=jax.ShapeDtypeStruct((B, D), _f32),
    )(ea, eb, l1a, l1b, lb1, l2p, lb2)
    return out[:, :1]


# speedup vs baseline: 1.0207x; 1.0066x over previous
import functools

import jax
import jax.numpy as jnp
from jax import lax
from jax.experimental import pallas as pl
from jax.experimental.pallas import tpu as pltpu
from jax.experimental.pallas import tpu_sc as plsc

N = 10000
E = 160000
D = 128
NSTEPS = 16
B = 128

NC, NS = 2, 16
NW = NC * NS
EPW = E // NW
KE = 40
NCH = EPW // KE
KZ = 200
NZCH = N // KZ
KG = 40
NGCH = N // KG
KH = 16

_f32 = jnp.float32


@functools.cache
def _mesh():
    return plsc.VectorSubcoreMesh(core_axis_name="c", subcore_axis_name="s")


def _zero_acc_start(zeros_hbm, s_sh, s, zsem):
    def cp(j, _):
        k = s + NS * j

        @pl.when(k < NZCH)
        def _():
            off = k * KZ
            pltpu.async_copy(zeros_hbm.at[pl.ds(off, KZ)],
                             s_sh.at[pl.ds(off, KZ)], zsem)

        return 0

    lax.fori_loop(0, (NZCH + NS - 1) // NS, cp, 0)


def _zero_acc_wait(zeros_hbm, s_sh, s, zsem):
    def cp(j, _):
        k = s + NS * j

        @pl.when(k < NZCH)
        def _():
            off = k * KZ
            pltpu.make_async_copy(zeros_hbm.at[pl.ds(off, KZ)],
                                  s_sh.at[pl.ds(off, KZ)], zsem).wait()

        return 0

    lax.fori_loop(0, (NZCH + NS - 1) // NS, cp, 0)


def _flush_acc(s_sh, out_hbm, c, s, zsem):
    def cp(issue, j, _):
        k = s + NS * j

        @pl.when(k < NZCH)
        def _():
            off = k * KZ
            d = pltpu.make_async_copy(s_sh.at[pl.ds(off, KZ)],
                                      out_hbm.at[c, pl.ds(off, KZ)], zsem)
            if issue:
                d.start()
            else:
                d.wait()

        return 0

    lax.fori_loop(0, (NZCH + NS - 1) // NS, functools.partial(cp, True), 0)
    lax.fori_loop(0, (NZCH + NS - 1) // NS, functools.partial(cp, False), 0)


@functools.cache
def _edge_pass_kernel():
    return functools.partial(
        pl.kernel,
        out_type=jax.ShapeDtypeStruct((NC, N, D), _f32),
        mesh=_mesh(),
        scratch_types=(
            [pltpu.VMEM((KE,), jnp.int32)] * 8
            + [pltpu.VMEM((KE, D), _f32)] * 6
            + [pltpu.VMEM_SHARED((N, D), _f32)]
            + [pltpu.SemaphoreType.DMA] * 11
        ),
    )(_edge_pass_body)


def _edge_pass_body(p_hbm, q_hbm, src_hbm, dst_hbm, zeros_hbm, out_hbm,
                    is0, is1, is2, is3, id0, id1, id2, id3,
                    pr0, pr1, qr0, qr1, qr2, qr3,
                    s_sh, si0, si1, si2, si3, gs0, gs1,
                    ss0, ss1, ss2, ss3, zsem):
    c = lax.axis_index("c")
    s = lax.axis_index("s")
    w = c * NS + s

    _zero_acc_start(zeros_hbm, s_sh, s, zsem)

    base_w = w * EPW
    idx_s, idx_d = (is0, is1, is2, is3), (id0, id1, id2, id3)
    pr, qr = (pr0, pr1), (qr0, qr1, qr2, qr3)
    isem = (si0, si1, si2, si3)
    gsem, ssem = (gs0, gs1), (ss0, ss1, ss2, ss3)

    def load_idx(ci, k4):
        base = base_w + ci * KE
        pltpu.async_copy(src_hbm.at[pl.ds(base, KE)], idx_s[k4], isem[k4])
        pltpu.async_copy(dst_hbm.at[pl.ds(base, KE)], idx_d[k4], isem[k4])

    def wait_idx(k4):
        pltpu.make_async_copy(src_hbm.at[pl.ds(0, KE)], idx_s[k4], isem[k4]).wait()
        pltpu.make_async_copy(src_hbm.at[pl.ds(0, KE)], idx_d[k4], isem[k4]).wait()

    def gather_start(k4, k2):
        pltpu.async_copy(p_hbm.at[idx_s[k4]], pr[k2], gsem[k2])
        pltpu.async_copy(q_hbm.at[idx_d[k4]], qr[k4], gsem[k2])

    def wait_gathers(k2):
        pltpu.make_async_copy(p_hbm.at[pl.ds(0, KE)], pr[k2], gsem[k2]).wait()
        pltpu.make_async_copy(q_hbm.at[pl.ds(0, KE)], qr[0], gsem[k2]).wait()

    def drain_scatter(k4):
        pltpu.make_async_copy(p_hbm.at[pl.ds(0, KE)], qr[k4], ssem[k4]).wait()

    def compute(k4, k2):
        def row(r, _):
            r0 = 2 * r
            for rr in (r0, r0 + 1):
                for cc in range(D // 16):
                    sl = pl.ds(cc * 16, 16)
                    qr[k4][rr, sl] = jnp.maximum(
                        pr[k2][rr, sl] + qr[k4][rr, sl], 0.0)
            return 0

        lax.fori_loop(0, KE // 2, row, 0)

    def scatter_start(k4):
        pltpu.async_copy(qr[k4], s_sh.at[idx_s[k4]], ssem[k4], add=True)

    load_idx(0, 0)
    load_idx(1, 1)
    wait_idx(0)
    gather_start(0, 0)
    wait_idx(1)
    gather_start(1, 1)
    _zero_acc_wait(zeros_hbm, s_sh, s, zsem)
    plsc.subcore_barrier()

    def piped(g, _):
        for b in range(4):
            ci = 4 * g + b
            k4, k2 = b, b % 2

            @pl.when(ci >= 2)
            def _():
                drain_scatter((k4 + 2) % 4)

            @pl.when(ci < NCH - 2)
            def _():
                load_idx(ci + 2, (k4 + 2) % 4)

            wait_gathers(k2)
            compute(k4, k2)

            @pl.when(ci < NCH - 2)
            def _():
                wait_idx((k4 + 2) % 4)
                gather_start((k4 + 2) % 4, k2)

            scatter_start(k4)
        return 0

    lax.fori_loop(0, NCH // 4, piped, 0)

    ci = NCH - 1
    k4, k2 = ci % 4, ci % 2
    drain_scatter((k4 + 2) % 4)
    wait_gathers(k2)
    compute(k4, k2)
    scatter_start(k4)
    drain_scatter((NCH - 2) % 4)
    drain_scatter((NCH - 1) % 4)

    plsc.subcore_barrier()
    _flush_acc(s_sh, out_hbm, c, s, zsem)


@functools.cache
def _deg_pass_kernel():
    return functools.partial(
        pl.kernel,
        out_type=jax.ShapeDtypeStruct((NC, N, D), _f32),
        mesh=_mesh(),
        scratch_types=[
            pltpu.VMEM((KE,), jnp.int32),
            pltpu.VMEM((KE, D), _f32),
            pltpu.VMEM_SHARED((N, D), _f32),
            pltpu.SemaphoreType.DMA,
        ],
    )(_deg_pass_body)


def _deg_pass_body(src_hbm, zeros_hbm, out_hbm, idx_s, ones_b, s_sh, zsem):
    c = lax.axis_index("c")
    s = lax.axis_index("s")
    w = c * NS + s

    _zero_acc_start(zeros_hbm, s_sh, s, zsem)
    one = jnp.full((16,), 1.0, _f32)

    def row(r, _):
        for cc in range(D // 16):
            ones_b[r, pl.ds(cc * 16, 16)] = one
        return 0

    lax.fori_loop(0, KE, row, 0)
    _zero_acc_wait(zeros_hbm, s_sh, s, zsem)
    plsc.subcore_barrier()

    base_w = w * EPW

    def chunk(i, _):
        base = base_w + i * KE
        pltpu.sync_copy(src_hbm.at[pl.ds(base, KE)], idx_s)
        pltpu.sync_copy(ones_b, s_sh.at[idx_s], add=True)
        return 0

    lax.fori_loop(0, NCH, chunk, 0)
    plsc.subcore_barrier()
    _flush_acc(s_sh, out_hbm, c, s, zsem)


@functools.cache
def _gather3_kernel():
    return functools.partial(
        pl.kernel,
        out_type=[jax.ShapeDtypeStruct((N, D), _f32)] * 3,
        mesh=_mesh(),
        scratch_types=[
            pltpu.VMEM((KG,), jnp.int32),
            pltpu.VMEM((KG, D), _f32),
            pltpu.SemaphoreType.DMA,
        ],
    )(_gather3_body)


def _gather3_body(t1, t2, t3, idx_hbm, o1, o2, o3, idxv, buf, sem):
    c = lax.axis_index("c")
    s = lax.axis_index("s")
    w = c * NS + s

    def rnd(i, _):
        cid = w + i * NW

        @pl.when(cid < NGCH)
        def _():
            base = cid * KG
            pltpu.sync_copy(idx_hbm.at[pl.ds(base, KG)], idxv)
            for tbl, out in ((t1, o1), (t2, o2), (t3, o3)):
                pltpu.async_copy(tbl.at[idxv], buf, sem).wait()
                pltpu.sync_copy(buf, out.at[pl.ds(base, KG)])

        return 0

    lax.fori_loop(0, (NGCH + NW - 1) // NW, rnd, 0)


@functools.cache
def _heads_gather_kernel():
    return functools.partial(
        pl.kernel,
        out_type=[jax.ShapeDtypeStruct((B, D), _f32)] * 2,
        mesh=_mesh(),
        scratch_types=[
            pltpu.VMEM((KH,), jnp.int32),
            pltpu.VMEM((KH, D), _f32),
            pltpu.SemaphoreType.DMA,
        ],
    )(_heads_gather_body)


def _heads_gather_body(xa, ha, xb, hb, oa, ob, idxv, buf, sem):
    c = lax.axis_index("c")
    s = lax.axis_index("s")
    base = s * KH
    active = s < (B // KH)

    @pl.when(jnp.logical_and(c == 0, active))
    def _():
        pltpu.sync_copy(ha.at[pl.ds(base, KH)], idxv)
        pltpu.async_copy(xa.at[idxv], buf, sem).wait()
        pltpu.sync_copy(buf, oa.at[pl.ds(base, KH)])

    @pl.when(jnp.logical_and(c == 1, active))
    def _():
        pltpu.sync_copy(hb.at[pl.ds(base, KH)], idxv)
        pltpu.async_copy(xb.at[idxv], buf, sem).wait()
        pltpu.sync_copy(buf, ob.at[pl.ds(base, KH)])


def _prep_body(emb_r, w1a_r, w1b_r, b1_r, pw1b_r, pb1_r, e1_r, e2_r, e3_r):
    x = emb_r[...]
    e1_r[...] = jnp.dot(x, w1a_r[...], preferred_element_type=_f32)
    e2_r[...] = jnp.dot(x, w1b_r[...], preferred_element_type=_f32) + b1_r[...]
    e3_r[...] = jnp.dot(x, pw1b_r[...], preferred_element_type=_f32) + pb1_r[...]


def _step_body(s_r, sd_r, c0_r, w2_r, b2_r, pw1a_r, pw2_r, pb2_r,
               w1a_r, w1b_r, b1_r, p_r, q_r, x_r):
    S = s_r[0] + s_r[1]
    deg = sd_r[0] + sd_r[1]
    agg = jnp.dot(S, w2_r[...], preferred_element_type=_f32) + deg * b2_r[...]
    t = jnp.maximum(jnp.dot(agg, pw1a_r[...], preferred_element_type=_f32)
                    + c0_r[...], 0.0)
    x = jnp.dot(t, pw2_r[...], preferred_element_type=_f32) + pb2_r[...]
    x_r[...] = x
    p_r[...] = jnp.dot(x, w1a_r[...], preferred_element_type=_f32)
    q_r[...] = jnp.dot(x, w1b_r[...], preferred_element_type=_f32) + b1_r[...]


def _logits_body(ea_r, eb_r, l1a_r, l1b_r, lb1_r, l2_r, lb2_r, o_r):
    h = jnp.maximum(jnp.dot(ea_r[...], l1a_r[...], preferred_element_type=_f32)
                    + jnp.dot(eb_r[...], l1b_r[...], preferred_element_type=_f32)
                    + lb1_r[...], 0.0)
    o_r[...] = jnp.dot(h, l2_r[...], preferred_element_type=_f32) + lb2_r[...]


_BM = 1000


def _tc_step(S, Sd, C0, w2, b2, pw1a, pw2, pb2, w1a, w1b, b1):
    full2 = pl.BlockSpec((D, D), lambda i: (0, 0))
    bias = pl.BlockSpec((1, D), lambda i: (0, 0))
    blk = pl.BlockSpec((_BM, D), lambda i: (i, 0))
    acc = pl.BlockSpec((NC, _BM, D), lambda i: (0, i, 0))
    return pl.pallas_call(
        _step_body,
        grid=(N // _BM,),
        in_specs=[acc, acc, blk, full2, bias, full2, full2, bias,
                  full2, full2, bias],
        out_specs=[blk, blk, blk],
        out_shape=[jax.ShapeDtypeStruct((N, D), _f32)] * 3,
    )(S, Sd, C0, w2, b2, pw1a, pw2, pb2, w1a, w1b, b1)


def kernel(nodes_a, edges_a, heads_a, nodes_b, edges_b, heads_b,
           emb, msg_W1, msg_b1, msg_W2, msg_b2,
           post_W1, post_b1, post_W2, post_b2,
           logit_W1, logit_b1, logit_W2, logit_b2):
    i32 = jnp.int32
    w1a, w1b = msg_W1[:D], msg_W1[D:2 * D]
    pw1a, pw1b = post_W1[:D], post_W1[D:]
    l1a, l1b = logit_W1[:D], logit_W1[D:]
    b1 = msg_b1.reshape(1, D)
    b2 = msg_b2.reshape(1, D)
    pb1 = post_b1.reshape(1, D)
    pb2 = post_b2.reshape(1, D)
    lb1 = logit_b1.reshape(1, D)
    lb2 = jnp.broadcast_to(logit_b2.reshape(1, 1), (1, D))
    l2p = jnp.pad(logit_W2, ((0, 0), (0, D - 1)))

    e1, e2, e3 = pl.pallas_call(
        _prep_body,
        out_shape=[jax.ShapeDtypeStruct((emb.shape[0], D), _f32)] * 3,
    )(emb, w1a, w1b, b1, pw1b, pb1)

    def encode(nodes, edges, heads_unused):
        nodes = nodes.astype(i32)
        src = edges[:, 0].astype(i32)
        dst = edges[:, 1].astype(i32)
        zeros = jnp.zeros((N, D), _f32)
        P, Q, C0 = _gather3_kernel()(e1, e2, e3, nodes)
        Sd = _deg_pass_kernel()(src, zeros)
        x = None
        for _ in range(NSTEPS):
            S = _edge_pass_kernel()(P, Q, src, dst, zeros)
            P, Q, x = _tc_step(S, Sd, C0, msg_W2, b2, pw1a, post_W2, pb2,
                               w1a, w1b, b1)
        return x

    xa = encode(nodes_a, edges_a, heads_a)
    xb = encode(nodes_b, edges_b, heads_b)
    ea, eb = _heads_gather_kernel()(xa, heads_a.astype(i32),
                                    xb, heads_b.astype(i32))

    out = pl.pallas_call(
        _logits_body,
        out_shape=jax.ShapeDtypeStruct((B, D), _f32),
    )(ea, eb, l1a, l1b, lb1, l2p, lb2)
    return out[:, :1]
